# baseline (speedup 1.0000x reference)
import functools

import jax
import jax.numpy as jnp
import numpy as np
from jax import lax
from jax.experimental import pallas as pl
from jax.experimental.pallas import tpu as pltpu
from jax.experimental.pallas import tpu_sc as plsc

N_NODES = 10000
N_EDGES = 320000
DIM = 128
D_EDGE_DIM = 16
NC = 2
NS = 16
NW = NC * NS
CH = 64
NCHUNK = N_EDGES // CH
CPT = 160
CPT_A = 128
NCHUNK_PAD = NW * CPT
N_PAD = 10240
ROWS_PER_TILE = N_PAD // NS

_EPERM = np.empty((DIM,), dtype=np.int32)
for _j in range(DIM // 32):
    for _k in range(16):
        _EPERM[32 * _j + 2 * _k] = 32 * _j + _k
        _EPERM[32 * _j + 2 * _k + 1] = 32 * _j + 16 + _k


def _matmul_body(x_ref, w_ref, b_ref, o_ref):
    o_ref[...] = (
        jnp.dot(x_ref[...], w_ref[...], preferred_element_type=jnp.float32)
        + b_ref[...]
    )


def _matmul_epack_body(x_ref, w_ref, b_ref, o_ref):
    r = (jnp.dot(x_ref[...], w_ref[...], preferred_element_type=jnp.float32)
         + b_ref[...])
    rq = r.astype(jnp.bfloat16).astype(jnp.float32)
    u = lax.bitcast_convert_type(rq, jnp.int32).reshape(-1, 2, DIM)
    o_ref[...] = (u[:, 1, :] & jnp.int32(-65536)) | (
        (u[:, 0, :] >> 16) & jnp.int32(0xFFFF))


def _mm(x, w, b, block_rows):
    n, k = x.shape
    m = w.shape[1]
    grid = n // block_rows
    return pl.pallas_call(
        _matmul_body,
        grid=(grid,),
        in_specs=[
            pl.BlockSpec((block_rows, k), lambda i: (i, 0)),
            pl.BlockSpec((k, m), lambda i: (0, 0)),
            pl.BlockSpec((1, m), lambda i: (0, 0)),
        ],
        out_specs=pl.BlockSpec((block_rows, m), lambda i: (i, 0)),
        out_shape=jax.ShapeDtypeStruct((n, m), jnp.float32),
    )(x, w, b)


def _mp_body(h0_hbm, e_hbm, src_hbm, dst_hbm, out_hbm,
             src_v, dst_v, rows_v, e_v, acc_sh, esem, gsem, dsem, ssem,
             csem):
    cid = lax.axis_index("c")
    sid = lax.axis_index("s")
    wid = sid * NC + cid
    base = wid * CPT

    def zbody(r, _):
        for j in range(DIM // 16):
            rows_v[0][r, pl.ds(j * 16, 16)] = jnp.zeros((16,), jnp.float32)
        return 0
    lax.fori_loop(0, CH, zbody, 0)
    for t in range(ROWS_PER_TILE // CH):
        pltpu.sync_copy(rows_v[0],
                        acc_sh.at[pl.ds(sid * ROWS_PER_TILE + t * CH, CH)])
    plsc.subcore_barrier()

    def prefetch(j, buf):
        @pl.when((j >= 2) & (base + j - 2 < NCHUNK) & (j - 2 < CPT))
        def _():
            pltpu.make_async_copy(rows_v[buf], acc_sh.at[dst_v[buf]],
                                  csem[buf]).wait()

        @pl.when((j < CPT) & (base + j < NCHUNK))
        def _():
            off = (base + j) * (CH // 2)
            pltpu.async_copy(e_hbm.at[pl.ds(off, CH // 2)], e_v[buf],
                             esem[buf])
            pltpu.async_copy(dst_hbm.at[base + j], dst_v[buf], dsem[buf])
            pltpu.async_copy(src_hbm.at[base + j], src_v[buf], ssem[buf])

    def gather(j, buf):
        @pl.when((j < CPT) & (base + j < NCHUNK))
        def _():
            pltpu.make_async_copy(src_hbm.at[base + j], src_v[buf],
                                  ssem[buf]).wait()
            pltpu.async_copy(h0_hbm.at[src_v[buf]], rows_v[buf], gsem[buf])

    def process(i, buf):
        @pl.when(base + i < NCHUNK)
        def _():
            off = (base + i) * (CH // 2)
            pltpu.make_async_copy(e_hbm.at[pl.ds(off, CH // 2)], e_v[buf],
                                  esem[buf]).wait()
            pltpu.make_async_copy(dst_hbm.at[base + i], dst_v[buf],
                                  dsem[buf]).wait()
            pltpu.make_async_copy(h0_hbm.at[src_v[buf]], rows_v[buf],
                                  gsem[buf]).wait()

            def rbody(r2, _):
                for h in range(DIM // 16):
                    s = pl.ds(16 * h, 16)
                    v = e_v[buf][r2, s]
                    e0 = lax.bitcast_convert_type(v << 16, jnp.float32)
                    e1 = lax.bitcast_convert_type(v & jnp.int32(-65536),
                                                  jnp.float32)
                    rows_v[buf][2 * r2, s] = jnp.maximum(
                        rows_v[buf][2 * r2, s] + e0, 0.0)
                    rows_v[buf][2 * r2 + 1, s] = jnp.maximum(
                        rows_v[buf][2 * r2 + 1, s] + e1, 0.0)
                return 0
            lax.fori_loop(0, CH // 2, rbody, 0)
            pltpu.async_copy(rows_v[buf], acc_sh.at[dst_v[buf]], csem[buf],
                             add=True)

    prefetch(0, 0)
    gather(0, 0)

    def pair_body(i2, _):
        for b in range(2):
            i = i2 * 2 + b
            prefetch(i + 1, 1 - b)
            gather(i + 1, 1 - b)
            process(i, b)
        return 0

    lax.fori_loop(0, CPT // 2, pair_body, 0)

    @pl.when(base + CPT - 1 < NCHUNK)
    def _():
        pltpu.make_async_copy(rows_v[1], acc_sh.at[dst_v[1]],
                              csem[1]).wait()

    plsc.subcore_barrier()
    pltpu.sync_copy(
        acc_sh.at[pl.ds(sid * ROWS_PER_TILE, ROWS_PER_TILE)],
        out_hbm.at[cid, pl.ds(sid * ROWS_PER_TILE, ROWS_PER_TILE)])


@functools.cache
def _message_passing():
    return functools.partial(
        pl.kernel,
        out_type=jax.ShapeDtypeStruct((NC, N_PAD, DIM), jnp.float32),
        mesh=plsc.VectorSubcoreMesh(
            core_axis_name="c", subcore_axis_name="s",
            num_cores=NC, num_subcores=NS),
        scratch_types=[
            [pltpu.VMEM((CH,), jnp.int32), pltpu.VMEM((CH,), jnp.int32)],
            [pltpu.VMEM((CH,), jnp.int32), pltpu.VMEM((CH,), jnp.int32)],
            [pltpu.VMEM((CH, DIM), jnp.float32),
             pltpu.VMEM((CH, DIM), jnp.float32)],
            [pltpu.VMEM((CH // 2, DIM), jnp.int32),
             pltpu.VMEM((CH // 2, DIM), jnp.int32)],
            pltpu.VMEM_SHARED((N_PAD, DIM), jnp.float32),
            [pltpu.SemaphoreType.DMA, pltpu.SemaphoreType.DMA],
            [pltpu.SemaphoreType.DMA, pltpu.SemaphoreType.DMA],
            [pltpu.SemaphoreType.DMA, pltpu.SemaphoreType.DMA],
            [pltpu.SemaphoreType.DMA, pltpu.SemaphoreType.DMA],
            [pltpu.SemaphoreType.DMA, pltpu.SemaphoreType.DMA],
        ],
    )(_mp_body)


def _mlp_body(h0_ref, a0_ref, a1_ref, w1_ref, b1_ref, w2_ref, b2_ref,
              o_ref, s_ref):
    i = pl.program_id(0)
    h = h0_ref[...] + a0_ref[...] + a1_ref[...]
    t = jnp.maximum(
        jnp.dot(h, w1_ref[...], preferred_element_type=jnp.float32)
        + b1_ref[...], 0.0)
    o = (jnp.dot(t, w2_ref[...], preferred_element_type=jnp.float32)
         + b2_ref[...])
    o_ref[...] = o

    @pl.when(i == 0)
    def _():
        s_ref[...] = jnp.zeros_like(s_ref)

    s_ref[0:1, :] += jnp.sum(o, axis=0, keepdims=True)
    s_ref[1:2, :] += jnp.sum(o * o, axis=0, keepdims=True)


def _bn_body(o_ref, s_ref, g_ref, b_ref, out_ref):
    mean = s_ref[0:1, :] * (1.0 / N_NODES)
    msq = s_ref[1:2, :] * (1.0 / N_NODES)
    var = msq - mean * mean
    inv = lax.rsqrt(var + 1e-5)
    out_ref[...] = (o_ref[...] - mean) * inv * g_ref[...] + b_ref[...]


def kernel(x, edge_index, edge_attr, W_enc, b_enc, W_edge, b_edge,
           W1, b1, W2, b2, gamma, beta):
    pad = NCHUNK_PAD - NCHUNK
    src = jnp.pad(edge_index[0].astype(jnp.int32).reshape(NCHUNK, CH),
                  ((0, pad), (0, 0)))
    dst = jnp.pad(edge_index[1].astype(jnp.int32).reshape(NCHUNK, CH),
                  ((0, pad), (0, 0)))

    h0 = _mm(x, W_enc, b_enc.reshape(1, -1), 2000)

    e = pl.pallas_call(
        _matmul_epack_body,
        grid=(125,),
        in_specs=[
            pl.BlockSpec((2560, D_EDGE_DIM), lambda i: (i, 0)),
            pl.BlockSpec((D_EDGE_DIM, DIM), lambda i: (0, 0)),
            pl.BlockSpec((1, DIM), lambda i: (0, 0)),
        ],
        out_specs=pl.BlockSpec((1280, DIM), lambda i: (i, 0)),
        out_shape=jax.ShapeDtypeStruct((N_EDGES // 2, DIM), jnp.int32),
    )(edge_attr, W_edge, b_edge.reshape(1, -1))

    aggr = _message_passing()(h0, e, src, dst)[:, :N_NODES, :]

    br = 2000
    grid = N_NODES // br
    mlp_out, stats = pl.pallas_call(
        _mlp_body,
        grid=(grid,),
        in_specs=[
            pl.BlockSpec((br, DIM), lambda i: (i, 0)),
            pl.BlockSpec((br, DIM), lambda i: (i, 0)),
            pl.BlockSpec((br, DIM), lambda i: (i, 0)),
            pl.BlockSpec((DIM, 2 * DIM), lambda i: (0, 0)),
            pl.BlockSpec((1, 2 * DIM), lambda i: (0, 0)),
            pl.BlockSpec((2 * DIM, DIM), lambda i: (0, 0)),
            pl.BlockSpec((1, DIM), lambda i: (0, 0)),
        ],
        out_specs=[
            pl.BlockSpec((br, DIM), lambda i: (i, 0)),
            pl.BlockSpec((2, DIM), lambda i: (0, 0)),
        ],
        out_shape=[
            jax.ShapeDtypeStruct((N_NODES, DIM), jnp.float32),
            jax.ShapeDtypeStruct((2, DIM), jnp.float32),
        ],
    )(h0, aggr[0], aggr[1], W1, b1.reshape(1, -1), W2, b2.reshape(1, -1))

    out = pl.pallas_call(
        _bn_body,
        grid=(grid,),
        in_specs=[
            pl.BlockSpec((br, DIM), lambda i: (i, 0)),
            pl.BlockSpec((2, DIM), lambda i: (0, 0)),
            pl.BlockSpec((1, DIM), lambda i: (0, 0)),
            pl.BlockSpec((1, DIM), lambda i: (0, 0)),
        ],
        out_specs=pl.BlockSpec((br, DIM), lambda i: (i, 0)),
        out_shape=jax.ShapeDtypeStruct((N_NODES, DIM), jnp.float32),
    )(mlp_out, stats, gamma.reshape(1, -1), beta.reshape(1, -1))
    return out


# speedup vs baseline: 1.3831x; 1.3831x over previous
import functools

import jax
import jax.numpy as jnp
from jax import lax
from jax.experimental import pallas as pl
from jax.experimental.pallas import tpu as pltpu
from jax.experimental.pallas import tpu_sc as plsc

N_NODES = 10000
N_EDGES = 320000
DIM = 128
D_EDGE_DIM = 16
NC = 2
NS = 16
NW = NC * NS
CH = 64
NCHUNK = N_EDGES // CH
CPT = 160
CPT_A = 128
NCHUNK_PAD = NW * CPT
N_PAD = 10240
ROWS_PER_TILE = N_PAD // NS

def _matmul_body(x_ref, w_ref, b_ref, o_ref):
    o_ref[...] = (
        jnp.dot(x_ref[...], w_ref[...], preferred_element_type=jnp.float32)
        + b_ref[...]
    )


def _mm(x, w, b, block_rows):
    n, k = x.shape
    m = w.shape[1]
    grid = n // block_rows
    return pl.pallas_call(
        _matmul_body,
        grid=(grid,),
        in_specs=[
            pl.BlockSpec((block_rows, k), lambda i: (i, 0)),
            pl.BlockSpec((k, m), lambda i: (0, 0)),
            pl.BlockSpec((1, m), lambda i: (0, 0)),
        ],
        out_specs=pl.BlockSpec((block_rows, m), lambda i: (i, 0)),
        out_shape=jax.ShapeDtypeStruct((n, m), jnp.float32),
    )(x, w, b)


def _mp_body(h0_hbm, e_hbm, src_hbm, dst_hbm, out_hbm,
             src_v, dst_v, rows_v, e_v, acc_sh, esem, gsem, dsem, ssem,
             csem):
    cid = lax.axis_index("c")
    sid = lax.axis_index("s")
    wid = sid * NC + cid
    base = wid * CPT

    def zbody(r, _):
        for j in range(DIM // 16):
            rows_v[0][r, pl.ds(j * 16, 16)] = jnp.zeros((16,), jnp.float32)
        return 0
    lax.fori_loop(0, CH, zbody, 0)
    for t in range(ROWS_PER_TILE // CH):
        pltpu.sync_copy(rows_v[0],
                        acc_sh.at[pl.ds(sid * ROWS_PER_TILE + t * CH, CH)])
    plsc.subcore_barrier()

    def prefetch(j, buf):
        @pl.when((j >= 2) & (base + j - 2 < NCHUNK) & (j - 2 < CPT))
        def _():
            pltpu.make_async_copy(rows_v[buf], acc_sh.at[dst_v[buf]],
                                  csem[buf]).wait()

        @pl.when((j < CPT) & (base + j < NCHUNK))
        def _():
            off = (base + j) * CH
            pltpu.async_copy(e_hbm.at[pl.ds(off, CH)], e_v[buf], esem[buf])
            pltpu.async_copy(dst_hbm.at[base + j], dst_v[buf], dsem[buf])
            pltpu.async_copy(src_hbm.at[base + j], src_v[buf], ssem[buf])

    def gather(j, buf):
        @pl.when((j < CPT) & (base + j < NCHUNK))
        def _():
            pltpu.make_async_copy(src_hbm.at[base + j], src_v[buf],
                                  ssem[buf]).wait()
            pltpu.async_copy(h0_hbm.at[src_v[buf]], rows_v[buf], gsem[buf])

    def process(i, buf):
        @pl.when(base + i < NCHUNK)
        def _():
            off = (base + i) * CH
            pltpu.make_async_copy(e_hbm.at[pl.ds(off, CH)], e_v[buf],
                                  esem[buf]).wait()
            pltpu.make_async_copy(dst_hbm.at[base + i], dst_v[buf],
                                  dsem[buf]).wait()
            pltpu.make_async_copy(h0_hbm.at[src_v[buf]], rows_v[buf],
                                  gsem[buf]).wait()

            def rbody(r2, _):
                for u in range(2):
                    r = r2 * 2 + u
                    for j in range(DIM // 16):
                        s = pl.ds(j * 16, 16)
                        rows_v[buf][r, s] = jnp.maximum(
                            rows_v[buf][r, s] + e_v[buf][r, s], 0.0)
                return 0
            lax.fori_loop(0, CH // 2, rbody, 0)
            pltpu.async_copy(rows_v[buf], acc_sh.at[dst_v[buf]], csem[buf],
                             add=True)

    prefetch(0, 0)
    gather(0, 0)

    def pair_body(i2, _):
        for b in range(2):
            i = i2 * 2 + b
            prefetch(i + 1, 1 - b)
            gather(i + 1, 1 - b)
            process(i, b)
        return 0

    lax.fori_loop(0, CPT // 2, pair_body, 0)

    @pl.when(base + CPT - 1 < NCHUNK)
    def _():
        pltpu.make_async_copy(rows_v[1], acc_sh.at[dst_v[1]],
                              csem[1]).wait()

    plsc.subcore_barrier()
    pltpu.sync_copy(
        acc_sh.at[pl.ds(sid * ROWS_PER_TILE, ROWS_PER_TILE)],
        out_hbm.at[cid, pl.ds(sid * ROWS_PER_TILE, ROWS_PER_TILE)])


@functools.cache
def _message_passing():
    return functools.partial(
        pl.kernel,
        out_type=jax.ShapeDtypeStruct((NC, N_PAD, DIM), jnp.float32),
        mesh=plsc.VectorSubcoreMesh(
            core_axis_name="c", subcore_axis_name="s",
            num_cores=NC, num_subcores=NS),
        scratch_types=[
            [pltpu.VMEM((CH,), jnp.int32), pltpu.VMEM((CH,), jnp.int32)],
            [pltpu.VMEM((CH,), jnp.int32), pltpu.VMEM((CH,), jnp.int32)],
            [pltpu.VMEM((CH, DIM), jnp.float32),
             pltpu.VMEM((CH, DIM), jnp.float32)],
            [pltpu.VMEM((CH, DIM), jnp.float32),
             pltpu.VMEM((CH, DIM), jnp.float32)],
            pltpu.VMEM_SHARED((N_PAD, DIM), jnp.float32),
            [pltpu.SemaphoreType.DMA, pltpu.SemaphoreType.DMA],
            [pltpu.SemaphoreType.DMA, pltpu.SemaphoreType.DMA],
            [pltpu.SemaphoreType.DMA, pltpu.SemaphoreType.DMA],
            [pltpu.SemaphoreType.DMA, pltpu.SemaphoreType.DMA],
            [pltpu.SemaphoreType.DMA, pltpu.SemaphoreType.DMA],
        ],
    )(_mp_body)


def _mlp_body(h0_ref, a0_ref, a1_ref, w1_ref, b1_ref, w2_ref, b2_ref,
              o_ref, s_ref):
    i = pl.program_id(0)
    h = h0_ref[...] + a0_ref[...] + a1_ref[...]
    t = jnp.maximum(
        jnp.dot(h, w1_ref[...], preferred_element_type=jnp.float32)
        + b1_ref[...], 0.0)
    o = (jnp.dot(t, w2_ref[...], preferred_element_type=jnp.float32)
         + b2_ref[...])
    o_ref[...] = o

    @pl.when(i == 0)
    def _():
        s_ref[...] = jnp.zeros_like(s_ref)

    s_ref[0:1, :] += jnp.sum(o, axis=0, keepdims=True)
    s_ref[1:2, :] += jnp.sum(o * o, axis=0, keepdims=True)


def _bn_body(o_ref, s_ref, g_ref, b_ref, out_ref):
    mean = s_ref[0:1, :] * (1.0 / N_NODES)
    msq = s_ref[1:2, :] * (1.0 / N_NODES)
    var = msq - mean * mean
    inv = lax.rsqrt(var + 1e-5)
    out_ref[...] = (o_ref[...] - mean) * inv * g_ref[...] + b_ref[...]


def kernel(x, edge_index, edge_attr, W_enc, b_enc, W_edge, b_edge,
           W1, b1, W2, b2, gamma, beta):
    pad = NCHUNK_PAD - NCHUNK
    src = jnp.pad(edge_index[0].astype(jnp.int32).reshape(NCHUNK, CH),
                  ((0, pad), (0, 0)))
    dst = jnp.pad(edge_index[1].astype(jnp.int32).reshape(NCHUNK, CH),
                  ((0, pad), (0, 0)))

    h0 = _mm(x, W_enc, b_enc.reshape(1, -1), 2000)

    e = _mm(edge_attr, W_edge, b_edge.reshape(1, -1), 2560)

    aggr = _message_passing()(h0, e, src, dst)[:, :N_NODES, :]

    br = 2000
    grid = N_NODES // br
    mlp_out, stats = pl.pallas_call(
        _mlp_body,
        grid=(grid,),
        in_specs=[
            pl.BlockSpec((br, DIM), lambda i: (i, 0)),
            pl.BlockSpec((br, DIM), lambda i: (i, 0)),
            pl.BlockSpec((br, DIM), lambda i: (i, 0)),
            pl.BlockSpec((DIM, 2 * DIM), lambda i: (0, 0)),
            pl.BlockSpec((1, 2 * DIM), lambda i: (0, 0)),
            pl.BlockSpec((2 * DIM, DIM), lambda i: (0, 0)),
            pl.BlockSpec((1, DIM), lambda i: (0, 0)),
        ],
        out_specs=[
            pl.BlockSpec((br, DIM), lambda i: (i, 0)),
            pl.BlockSpec((2, DIM), lambda i: (0, 0)),
        ],
        out_shape=[
            jax.ShapeDtypeStruct((N_NODES, DIM), jnp.float32),
            jax.ShapeDtypeStruct((2, DIM), jnp.float32),
        ],
    )(h0, aggr[0], aggr[1], W1, b1.reshape(1, -1), W2, b2.reshape(1, -1))

    out = pl.pallas_call(
        _bn_body,
        grid=(grid,),
        in_specs=[
            pl.BlockSpec((br, DIM), lambda i: (i, 0)),
            pl.BlockSpec((2, DIM), lambda i: (0, 0)),
            pl.BlockSpec((1, DIM), lambda i: (0, 0)),
            pl.BlockSpec((1, DIM), lambda i: (0, 0)),
        ],
        out_specs=pl.BlockSpec((br, DIM), lambda i: (i, 0)),
        out_shape=jax.ShapeDtypeStruct((N_NODES, DIM), jnp.float32),
    )(mlp_out, stats, gamma.reshape(1, -1), beta.reshape(1, -1))
    return out


# speedup vs baseline: 1.4544x; 1.0515x over previous
import functools

import jax
import jax.numpy as jnp
from jax import lax
from jax.experimental import pallas as pl
from jax.experimental.pallas import tpu as pltpu
from jax.experimental.pallas import tpu_sc as plsc

N_NODES = 10000
N_EDGES = 320000
DIM = 128
D_EDGE_DIM = 16
NC = 2
NS = 16
NW = NC * NS
CH = 64
NCHUNK = N_EDGES // CH
CPT = 160
CPT_A = 128
NCHUNK_PAD = NW * CPT
N_PAD = 10240
ROWS_PER_TILE = N_PAD // NS

def _matmul_body(x_ref, w_ref, b_ref, o_ref):
    o_ref[...] = (
        jnp.dot(x_ref[...], w_ref[...], preferred_element_type=jnp.float32)
        + b_ref[...]
    )


def _mm(x, w, b, block_rows):
    n, k = x.shape
    m = w.shape[1]
    grid = n // block_rows
    return pl.pallas_call(
        _matmul_body,
        grid=(grid,),
        in_specs=[
            pl.BlockSpec((block_rows, k), lambda i: (i, 0)),
            pl.BlockSpec((k, m), lambda i: (0, 0)),
            pl.BlockSpec((1, m), lambda i: (0, 0)),
        ],
        out_specs=pl.BlockSpec((block_rows, m), lambda i: (i, 0)),
        out_shape=jax.ShapeDtypeStruct((n, m), jnp.float32),
    )(x, w, b)


def _mp_body(h0_hbm, e_hbm, src_hbm, dst_hbm, out_hbm,
             src_v, dst_v, rows_v, e_v, acc_sh, esem, gsem, dsem, ssem,
             csem):
    cid = lax.axis_index("c")
    sid = lax.axis_index("s")
    wid = sid * NC + cid
    base = wid * CPT

    def zbody(r, _):
        for j in range(DIM // 16):
            rows_v[0][r, pl.ds(j * 16, 16)] = jnp.zeros((16,), jnp.float32)
        return 0
    lax.fori_loop(0, CH, zbody, 0)
    for t in range(ROWS_PER_TILE // CH):
        pltpu.sync_copy(rows_v[0],
                        acc_sh.at[pl.ds(sid * ROWS_PER_TILE + t * CH, CH)])
    plsc.subcore_barrier()

    def prefetch(j, buf):
        @pl.when((j >= 2) & (base + j - 2 < NCHUNK) & (j - 2 < CPT))
        def _():
            pltpu.make_async_copy(rows_v[buf], acc_sh.at[dst_v[buf]],
                                  csem[buf]).wait()

        @pl.when((j < CPT) & (base + j < NCHUNK))
        def _():
            off = (base + j) * CH
            pltpu.async_copy(e_hbm.at[pl.ds(off, CH)], e_v[buf], esem[buf])
            pltpu.async_copy(dst_hbm.at[pl.ds(off, CH)], dst_v[buf],
                             dsem[buf])
            pltpu.async_copy(src_hbm.at[pl.ds(off, CH)], src_v[buf],
                             ssem[buf])

    def gather(j, buf):
        @pl.when((j < CPT) & (base + j < NCHUNK))
        def _():
            pltpu.make_async_copy(src_hbm.at[pl.ds((base + j) * CH, CH)],
                                  src_v[buf], ssem[buf]).wait()
            pltpu.async_copy(h0_hbm.at[src_v[buf]], rows_v[buf], gsem[buf])

    def process(i, buf):
        @pl.when(base + i < NCHUNK)
        def _():
            off = (base + i) * CH
            pltpu.make_async_copy(e_hbm.at[pl.ds(off, CH)], e_v[buf],
                                  esem[buf]).wait()
            pltpu.make_async_copy(dst_hbm.at[pl.ds(off, CH)], dst_v[buf],
                                  dsem[buf]).wait()
            pltpu.make_async_copy(h0_hbm.at[src_v[buf]], rows_v[buf],
                                  gsem[buf]).wait()

            def rbody(r2, _):
                for u in range(2):
                    r = r2 * 2 + u
                    for j in range(DIM // 16):
                        s = pl.ds(j * 16, 16)
                        rows_v[buf][r, s] = jnp.maximum(
                            rows_v[buf][r, s] + e_v[buf][r, s], 0.0)
                return 0
            lax.fori_loop(0, CH // 2, rbody, 0)
            pltpu.async_copy(rows_v[buf], acc_sh.at[dst_v[buf]], csem[buf],
                             add=True)

    prefetch(0, 0)
    gather(0, 0)

    def pair_body(i2, _):
        for b in range(2):
            i = i2 * 2 + b
            prefetch(i + 1, 1 - b)
            gather(i + 1, 1 - b)
            process(i, b)
        return 0

    lax.fori_loop(0, CPT // 2, pair_body, 0)

    @pl.when(base + CPT - 1 < NCHUNK)
    def _():
        pltpu.make_async_copy(rows_v[1], acc_sh.at[dst_v[1]],
                              csem[1]).wait()

    plsc.subcore_barrier()
    pltpu.sync_copy(
        acc_sh.at[pl.ds(sid * ROWS_PER_TILE, ROWS_PER_TILE)],
        out_hbm.at[cid, pl.ds(sid * ROWS_PER_TILE, ROWS_PER_TILE)])


@functools.cache
def _message_passing():
    return functools.partial(
        pl.kernel,
        out_type=jax.ShapeDtypeStruct((NC, N_PAD, DIM), jnp.float32),
        mesh=plsc.VectorSubcoreMesh(
            core_axis_name="c", subcore_axis_name="s",
            num_cores=NC, num_subcores=NS),
        scratch_types=[
            [pltpu.VMEM((CH,), jnp.int32), pltpu.VMEM((CH,), jnp.int32)],
            [pltpu.VMEM((CH,), jnp.int32), pltpu.VMEM((CH,), jnp.int32)],
            [pltpu.VMEM((CH, DIM), jnp.float32),
             pltpu.VMEM((CH, DIM), jnp.float32)],
            [pltpu.VMEM((CH, DIM), jnp.float32),
             pltpu.VMEM((CH, DIM), jnp.float32)],
            pltpu.VMEM_SHARED((N_PAD, DIM), jnp.float32),
            [pltpu.SemaphoreType.DMA, pltpu.SemaphoreType.DMA],
            [pltpu.SemaphoreType.DMA, pltpu.SemaphoreType.DMA],
            [pltpu.SemaphoreType.DMA, pltpu.SemaphoreType.DMA],
            [pltpu.SemaphoreType.DMA, pltpu.SemaphoreType.DMA],
            [pltpu.SemaphoreType.DMA, pltpu.SemaphoreType.DMA],
        ],
    )(_mp_body)


def _mlp_body(h0_ref, a0_ref, a1_ref, w1_ref, b1_ref, w2_ref, b2_ref,
              o_ref, s_ref):
    i = pl.program_id(0)
    h = h0_ref[...] + a0_ref[0] + a1_ref[0]
    t = jnp.maximum(
        jnp.dot(h, w1_ref[...], preferred_element_type=jnp.float32)
        + b1_ref[...], 0.0)
    o = (jnp.dot(t, w2_ref[...], preferred_element_type=jnp.float32)
         + b2_ref[...])
    o_ref[...] = o

    @pl.when(i == 0)
    def _():
        s_ref[...] = jnp.zeros_like(s_ref)

    s_ref[0:1, :] += jnp.sum(o, axis=0, keepdims=True)
    s_ref[1:2, :] += jnp.sum(o * o, axis=0, keepdims=True)


def _bn_body(o_ref, s_ref, g_ref, b_ref, out_ref):
    mean = s_ref[0:1, :] * (1.0 / N_NODES)
    msq = s_ref[1:2, :] * (1.0 / N_NODES)
    var = msq - mean * mean
    inv = lax.rsqrt(var + 1e-5)
    out_ref[...] = (o_ref[...] - mean) * inv * g_ref[...] + b_ref[...]


def kernel(x, edge_index, edge_attr, W_enc, b_enc, W_edge, b_edge,
           W1, b1, W2, b2, gamma, beta):
    src = edge_index[0].astype(jnp.int32)
    dst = edge_index[1].astype(jnp.int32)

    h0 = _mm(x, W_enc, b_enc.reshape(1, -1), 2000)

    e = _mm(edge_attr, W_edge, b_edge.reshape(1, -1), 2560)

    aggr = _message_passing()(h0, e, src, dst)

    br = 2000
    grid = N_NODES // br
    mlp_out, stats = pl.pallas_call(
        _mlp_body,
        grid=(grid,),
        in_specs=[
            pl.BlockSpec((br, DIM), lambda i: (i, 0)),
            pl.BlockSpec((1, br, DIM), lambda i: (0, i, 0)),
            pl.BlockSpec((1, br, DIM), lambda i: (1, i, 0)),
            pl.BlockSpec((DIM, 2 * DIM), lambda i: (0, 0)),
            pl.BlockSpec((1, 2 * DIM), lambda i: (0, 0)),
            pl.BlockSpec((2 * DIM, DIM), lambda i: (0, 0)),
            pl.BlockSpec((1, DIM), lambda i: (0, 0)),
        ],
        out_specs=[
            pl.BlockSpec((br, DIM), lambda i: (i, 0)),
            pl.BlockSpec((2, DIM), lambda i: (0, 0)),
        ],
        out_shape=[
            jax.ShapeDtypeStruct((N_NODES, DIM), jnp.float32),
            jax.ShapeDtypeStruct((2, DIM), jnp.float32),
        ],
    )(h0, aggr, aggr, W1, b1.reshape(1, -1), W2, b2.reshape(1, -1))

    out = pl.pallas_call(
        _bn_body,
        grid=(grid,),
        in_specs=[
            pl.BlockSpec((br, DIM), lambda i: (i, 0)),
            pl.BlockSpec((2, DIM), lambda i: (0, 0)),
            pl.BlockSpec((1, DIM), lambda i: (0, 0)),
            pl.BlockSpec((1, DIM), lambda i: (0, 0)),
        ],
        out_specs=pl.BlockSpec((br, DIM), lambda i: (i, 0)),
        out_shape=jax.ShapeDtypeStruct((N_NODES, DIM), jnp.float32),
    )(mlp_out, stats, gamma.reshape(1, -1), beta.reshape(1, -1))
    return out
